# baseline (speedup 1.0000x reference)
import functools

import jax
import jax.numpy as jnp
from jax.experimental import pallas as pl

_N_COLS = 8192
_LANES = 128
_GROUPS = _N_COLS // _LANES


def _cumsum_body(x_ref, o_ref, *, block_rows):
    x = x_ref[...]
    xg = x.reshape(block_rows * _GROUPS, _LANES)

    li = jax.lax.broadcasted_iota(jnp.int32, (_LANES, _LANES), 0)
    lj = jax.lax.broadcasted_iota(jnp.int32, (_LANES, _LANES), 1)
    scan_mat = (li <= lj).astype(jnp.float32)

    y = jnp.dot(xg, scan_mat, preferred_element_type=jnp.float32)
    y = y.reshape(block_rows, _GROUPS, _LANES)

    totals = y[:, :, _LANES - 1]
    gi = jax.lax.broadcasted_iota(jnp.int32, (_GROUPS, _GROUPS), 0)
    gj = jax.lax.broadcasted_iota(jnp.int32, (_GROUPS, _GROUPS), 1)
    carry_mat = (gi < gj).astype(jnp.float32)
    carries = jnp.dot(totals, carry_mat, preferred_element_type=jnp.float32)

    o_ref[...] = (y + carries[:, :, None]).reshape(block_rows, _N_COLS)


@jax.jit
def kernel(x):
    n_rows, n_cols = x.shape
    block_rows = 128
    grid = (n_rows // block_rows,)
    return pl.pallas_call(
        functools.partial(_cumsum_body, block_rows=block_rows),
        grid=grid,
        in_specs=[pl.BlockSpec((block_rows, n_cols), lambda i: (i, 0))],
        out_specs=pl.BlockSpec((block_rows, n_cols), lambda i: (i, 0)),
        out_shape=jax.ShapeDtypeStruct((n_rows, n_cols), x.dtype),
    )(x)


# speedup vs baseline: 1.2440x; 1.2440x over previous
import functools

import jax
import jax.numpy as jnp
from jax.experimental import pallas as pl

_N_COLS = 8192
_LANES = 128
_GROUPS = _N_COLS // _LANES


def _cumsum_body(x_ref, o_ref, *, block_rows):
    li = jax.lax.broadcasted_iota(jnp.int32, (_LANES, _LANES), 0)
    lj = jax.lax.broadcasted_iota(jnp.int32, (_LANES, _LANES), 1)
    scan_mat = (li <= lj).astype(jnp.float32)

    carry = jnp.zeros((block_rows, 1), dtype=jnp.float32)
    for g in range(_GROUPS):
        xg = x_ref[:, g * _LANES:(g + 1) * _LANES]
        scan = jnp.dot(xg, scan_mat, preferred_element_type=jnp.float32)
        o_ref[:, g * _LANES:(g + 1) * _LANES] = scan + carry
        if g + 1 < _GROUPS:
            carry = carry + scan[:, _LANES - 1:_LANES]


@jax.jit
def kernel(x):
    n_rows, n_cols = x.shape
    block_rows = 256
    grid = (n_rows // block_rows,)
    return pl.pallas_call(
        functools.partial(_cumsum_body, block_rows=block_rows),
        grid=grid,
        in_specs=[pl.BlockSpec((block_rows, n_cols), lambda i: (i, 0))],
        out_specs=pl.BlockSpec((block_rows, n_cols), lambda i: (i, 0)),
        out_shape=jax.ShapeDtypeStruct((n_rows, n_cols), x.dtype),
    )(x)
